# baseline (speedup 1.0000x reference)
import functools

import jax
import jax.numpy as jnp
from jax import lax
from jax.experimental import pallas as pl
from jax.experimental.pallas import tpu as pltpu
from jax.experimental.pallas import tpu_sc as plsc

_BW = 128
_L = 16


@functools.lru_cache(maxsize=None)
def _build(B, S, D, NC, NS):
    NW = NC * NS
    n_idx = B // NW * S
    mesh = plsc.VectorSubcoreMesh(core_axis_name="c", subcore_axis_name="s")

    @functools.partial(
        pl.kernel,
        mesh=mesh,
        out_type=jax.ShapeDtypeStruct((S, D, B), jnp.float32),
        scratch_types=[
            pltpu.VMEM((n_idx,), jnp.int32),
            pltpu.VMEM((n_idx,), jnp.int32),
            pltpu.VMEM((2, _BW, 2 * D), jnp.float32),
            pltpu.VMEM((2, D, _BW), jnp.float32),
            pltpu.SemaphoreType.DMA((2,)),
            pltpu.SemaphoreType.DMA((2,)),
        ],
        compiler_params=pltpu.CompilerParams(
            use_tc_tiling_on_sc=True, needs_layout_passes=False
        ),
    )
    def k(xw_hbm, tbl2_hbm, out_hbm, idx_v, idx2_v, bufs, trbufs, gsem, osem):
        wid = lax.axis_index("s") * NC + lax.axis_index("c")
        b0 = wid * _BW
        pltpu.sync_copy(xw_hbm.at[wid], idx_v)

        iota = lax.iota(jnp.int32, _L)

        def shift_block(g, c):
            for u in range(8):
                off = g * 128 + u * _L
                idx2_v[pl.ds(off, _L)] = lax.shift_right_logical(
                    idx_v[pl.ds(off, _L)], 1
                )
            return c

        lax.fori_loop(0, S, shift_block, 0)

        def gather(j, s):
            pltpu.async_copy(
                tbl2_hbm.at[idx2_v.at[pl.ds(j * _BW, _BW)]],
                bufs.at[s],
                gsem.at[s],
            )

        def wait_gather(s):
            pltpu.make_async_copy(
                tbl2_hbm.at[idx2_v.at[pl.ds(0, _BW)]], bufs.at[s], gsem.at[s]
            ).wait()

        def put(j, s):
            pltpu.async_copy(
                trbufs.at[s], out_hbm.at[j, :, pl.ds(b0, _BW)], osem.at[s]
            )

        def wait_put(s):
            pltpu.make_async_copy(
                trbufs.at[s], out_hbm.at[0, :, pl.ds(b0, _BW)], osem.at[s]
            ).wait()

        buf_s = [bufs.at[0], bufs.at[1]]
        tr_s = [trbufs.at[0], trbufs.at[1]]

        def extract(j, s):
            for gi in range(_BW // _L):
                i0 = gi * _L
                v_vec = idx_v[pl.ds(j * _BW + i0, _L)]
                h_vec = (v_vec & 1) * D
                row_vec = i0 + iota

                def dbody(d8, c, _s=s, _i0=i0, _h=h_vec, _r=row_vec):
                    for u in range(4):
                        d = d8 * 4 + u
                        vals = plsc.load_gather(buf_s[_s], [_r, _h + d])
                        tr_s[_s][d, pl.ds(_i0, _L)] = vals
                    return c

                lax.fori_loop(0, D // 4, dbody, 0)

        def stage(j, s, first, refill):
            wait_gather(s)
            if not first:
                wait_put(s)
            extract(j, s)
            put(j, s)
            if refill:
                gather(j + 2, s)

        gather(0, 0)
        gather(1, 1)
        stage(0, 0, True, True)
        stage(1, 1, True, True)

        def body(g, c):
            stage(2 * g, 0, False, True)
            stage(2 * g + 1, 1, False, True)
            return c

        lax.fori_loop(1, S // 2 - 1, body, 0)
        stage(S - 2, 0, False, False)
        stage(S - 1, 1, False, False)
        wait_put(0)
        wait_put(1)

    return k


def kernel(X, table):
    B, S = X.shape
    V, D = table.shape
    info = plsc.get_sparse_core_info()
    NC, NS = info.num_cores, info.num_subcores
    NW = NC * NS
    Xw = X.reshape(NW, _BW, S).transpose(0, 2, 1).reshape(NW, _BW * S)
    tbl2 = table.reshape(V // 2, 2 * D)
    P = _build(B, S, D, NC, NS)(Xw, tbl2)
    return P.transpose(2, 0, 1)


# speedup vs baseline: 1.1577x; 1.1577x over previous
import functools

import jax
import jax.numpy as jnp
from jax import lax
from jax.experimental import pallas as pl
from jax.experimental.pallas import tpu as pltpu
from jax.experimental.pallas import tpu_sc as plsc

_BW = 128
_L = 16


@functools.lru_cache(maxsize=None)
def _build(B, S, D, NC, NS):
    NW = NC * NS
    n_idx = B // NW * S
    mesh = plsc.VectorSubcoreMesh(core_axis_name="c", subcore_axis_name="s")

    @functools.partial(
        pl.kernel,
        mesh=mesh,
        out_type=jax.ShapeDtypeStruct((S, D, B), jnp.float32),
        scratch_types=[
            pltpu.VMEM((n_idx,), jnp.int32),
            pltpu.VMEM((2, _BW, 2 * D), jnp.float32),
            pltpu.VMEM((2, D, _BW), jnp.float32),
            pltpu.SemaphoreType.DMA((2,)),
            pltpu.SemaphoreType.DMA((2,)),
        ],
        compiler_params=pltpu.CompilerParams(
            use_tc_tiling_on_sc=True, needs_layout_passes=False
        ),
    )
    def k(xw_hbm, tblp_hbm, out_hbm, idx_v, bufs, trbufs, gsem, osem):
        wid = lax.axis_index("s") * NC + lax.axis_index("c")
        b0 = wid * _BW
        pltpu.sync_copy(xw_hbm.at[wid], idx_v)

        iota = lax.iota(jnp.int32, _L)
        dvecs = [k16 * _L + iota for k16 in range(D // _L)]

        def gather(j, s):
            pltpu.async_copy(
                tblp_hbm.at[idx_v.at[pl.ds(j * _BW, _BW)]],
                bufs.at[s],
                gsem.at[s],
            )

        def wait_gather(s):
            pltpu.make_async_copy(
                tblp_hbm.at[idx_v.at[pl.ds(0, _BW)]], bufs.at[s], gsem.at[s]
            ).wait()

        def put(j, s):
            pltpu.async_copy(
                trbufs.at[s], out_hbm.at[j, :, pl.ds(b0, _BW)], osem.at[s]
            )

        def wait_put(s):
            pltpu.make_async_copy(
                trbufs.at[s], out_hbm.at[0, :, pl.ds(b0, _BW)], osem.at[s]
            ).wait()

        buf_s = [bufs.at[0], bufs.at[1]]
        tr_s = [trbufs.at[0], trbufs.at[1]]

        def extract(s):
            def rbody(r8, c, _s=s):
                for u in range(8):
                    i = r8 * 8 + u
                    i_vec = jnp.full((_L,), 0, jnp.int32) + i
                    for k16 in range(D // _L):
                        vals = buf_s[_s][i, pl.ds(k16 * _L, _L)]
                        plsc.store_scatter(
                            tr_s[_s], [dvecs[k16], i_vec], vals
                        )
                return c

            lax.fori_loop(0, _BW // 8, rbody, 0)

        def stage(j, s, first, refill):
            wait_gather(s)
            if not first:
                wait_put(s)
            extract(s)
            put(j, s)
            if refill:
                gather(j + 2, s)

        gather(0, 0)
        gather(1, 1)
        stage(0, 0, True, True)
        stage(1, 1, True, True)

        def body(g, c):
            stage(2 * g, 0, False, True)
            stage(2 * g + 1, 1, False, True)
            return c

        lax.fori_loop(1, S // 2 - 1, body, 0)
        stage(S - 2, 0, False, False)
        stage(S - 1, 1, False, False)
        wait_put(0)
        wait_put(1)

    return k


def kernel(X, table):
    B, S = X.shape
    V, D = table.shape
    info = plsc.get_sparse_core_info()
    NC, NS = info.num_cores, info.num_subcores
    NW = NC * NS
    Xw = X.reshape(NW, _BW, S).transpose(0, 2, 1).reshape(NW, _BW * S)
    tblp = jnp.pad(table, ((0, 0), (0, D)))
    P = _build(B, S, D, NC, NS)(Xw, tblp)
    return P.transpose(2, 0, 1)


# speedup vs baseline: 1.3303x; 1.1491x over previous
import functools

import jax
import jax.numpy as jnp
from jax import lax
from jax.experimental import pallas as pl
from jax.experimental.pallas import tpu as pltpu
from jax.experimental.pallas import tpu_sc as plsc

_BW = 128
_L = 16


@functools.lru_cache(maxsize=None)
def _build(B, S, D, NC, NS):
    NW = NC * NS
    n_idx = B // NW * S
    mesh = plsc.VectorSubcoreMesh(core_axis_name="c", subcore_axis_name="s")

    @functools.partial(
        pl.kernel,
        mesh=mesh,
        out_type=jax.ShapeDtypeStruct((S, D, B), jnp.float32),
        scratch_types=[
            pltpu.VMEM((n_idx,), jnp.int32),
            pltpu.VMEM((2, _BW, 2 * D), jnp.float32),
            pltpu.VMEM((2, D, _BW), jnp.float32),
            pltpu.SemaphoreType.DMA((2,)),
            pltpu.SemaphoreType.DMA((2,)),
        ],
        compiler_params=pltpu.CompilerParams(
            use_tc_tiling_on_sc=True, needs_layout_passes=False
        ),
    )
    def k(xw_hbm, tblp_hbm, out_hbm, idx_v, bufs, trbufs, gsem, osem):
        wid = lax.axis_index("s") * NC + lax.axis_index("c")
        b0 = wid * _BW
        pltpu.sync_copy(xw_hbm.at[wid], idx_v)

        iota = lax.iota(jnp.int32, _L)
        diags = [(iota + r) & (_L - 1) for r in range(_L)]

        def gather(j, s):
            pltpu.async_copy(
                tblp_hbm.at[idx_v.at[pl.ds(j * _BW, _BW)]],
                bufs.at[s],
                gsem.at[s],
            )

        def wait_gather(s):
            pltpu.make_async_copy(
                tblp_hbm.at[idx_v.at[pl.ds(0, _BW)]], bufs.at[s], gsem.at[s]
            ).wait()

        def put(j, s):
            pltpu.async_copy(
                trbufs.at[s], out_hbm.at[j, :, pl.ds(b0, _BW)], osem.at[s]
            )

        def wait_put(s):
            pltpu.make_async_copy(
                trbufs.at[s], out_hbm.at[0, :, pl.ds(b0, _BW)], osem.at[s]
            ).wait()

        buf_s = [bufs.at[0], bufs.at[1]]
        tr_s = [trbufs.at[0], trbufs.at[1]]

        def extract(s):
            def gbody(gi, c, _s=s):
                row_vec = gi * _L + iota
                for k16 in range(D // _L):
                    for r in range(_L):
                        col_vec = k16 * _L + diags[r]
                        vals = plsc.load_gather(buf_s[_s], [row_vec, col_vec])
                        plsc.store_scatter(tr_s[_s], [col_vec, row_vec], vals)
                return c

            lax.fori_loop(0, _BW // _L, gbody, 0)

        def stage(j, s, first, refill):
            wait_gather(s)
            if not first:
                wait_put(s)
            extract(s)
            put(j, s)
            if refill:
                gather(j + 2, s)

        gather(0, 0)
        gather(1, 1)
        stage(0, 0, True, True)
        stage(1, 1, True, True)

        def body(g, c):
            stage(2 * g, 0, False, True)
            stage(2 * g + 1, 1, False, True)
            return c

        lax.fori_loop(1, S // 2 - 1, body, 0)
        stage(S - 2, 0, False, False)
        stage(S - 1, 1, False, False)
        wait_put(0)
        wait_put(1)

    return k


def kernel(X, table):
    B, S = X.shape
    V, D = table.shape
    info = plsc.get_sparse_core_info()
    NC, NS = info.num_cores, info.num_subcores
    NW = NC * NS
    Xw = X.reshape(NW, _BW, S).transpose(0, 2, 1).reshape(NW, _BW * S)
    tblp = jnp.pad(table, ((0, 0), (0, D)))
    P = _build(B, S, D, NC, NS)(Xw, tblp)
    return P.transpose(2, 0, 1)
